# baseline (speedup 1.0000x reference)
import functools

import jax
import jax.numpy as jnp
from jax import lax
from jax.experimental import pallas as pl
from jax.experimental.pallas import tpu as pltpu
from jax.experimental.pallas import tpu_sc as plsc

_mesh = plsc.VectorSubcoreMesh(core_axis_name="c", subcore_axis_name="s")


@functools.partial(
    pl.kernel,
    mesh=_mesh,
    out_type=jax.ShapeDtypeStruct((100,), jnp.float32),
    scratch_types=[
        pltpu.VMEM((32,), jnp.float32),
        pltpu.VMEM((112,), jnp.float32),
    ],
)
def _scatter_double(in_hbm, out_hbm, in_v, out_v):
    cid = lax.axis_index("c")
    sid = lax.axis_index("s")

    @pl.when(jnp.logical_and(cid == 0, sid == 0))
    def _():
        pltpu.sync_copy(in_hbm.at[pl.ds(0, 32)], in_v)
        out_v[pl.ds(0, 16)] = in_v[pl.ds(0, 16)] * 2.0
        lane = lax.iota(jnp.int32, 16)
        out_v[pl.ds(16, 16)] = jnp.where(
            lane < 10, in_v[pl.ds(16, 16)] * 2.0, 0.0
        )
        zeros = jnp.zeros((16,), jnp.float32)
        for j in range(2, 7):
            out_v[pl.ds(j * 16, 16)] = zeros
        pltpu.sync_copy(out_v.at[pl.ds(0, 100)], out_hbm)


def kernel(primals_1):
    return _scatter_double(primals_1)


# speedup vs baseline: 1.0900x; 1.0900x over previous
import functools

import jax
import jax.numpy as jnp
from jax import lax
from jax.experimental import pallas as pl
from jax.experimental.pallas import tpu as pltpu
from jax.experimental.pallas import tpu_sc as plsc

_mesh = plsc.VectorSubcoreMesh(
    core_axis_name="c", subcore_axis_name="s", num_cores=1
)


@functools.partial(
    pl.kernel,
    mesh=_mesh,
    out_type=jax.ShapeDtypeStruct((100,), jnp.float32),
    scratch_types=[
        pltpu.VMEM((32,), jnp.float32),
        pltpu.VMEM((112,), jnp.float32),
    ],
)
def _scatter_double(in_hbm, out_hbm, in_v, out_v):
    cid = lax.axis_index("c")
    sid = lax.axis_index("s")

    @pl.when(jnp.logical_and(cid == 0, sid == 0))
    def _():
        pltpu.sync_copy(in_hbm.at[pl.ds(0, 32)], in_v)
        out_v[pl.ds(0, 16)] = in_v[pl.ds(0, 16)] * 2.0
        lane = lax.iota(jnp.int32, 16)
        out_v[pl.ds(16, 16)] = jnp.where(
            lane < 10, in_v[pl.ds(16, 16)] * 2.0, 0.0
        )
        zeros = jnp.zeros((16,), jnp.float32)
        for j in range(2, 7):
            out_v[pl.ds(j * 16, 16)] = zeros
        pltpu.sync_copy(out_v.at[pl.ds(0, 100)], out_hbm)


def kernel(primals_1):
    return _scatter_double(primals_1)
